# baseline (speedup 1.0000x reference)
import functools

import jax
import jax.numpy as jnp
from jax import lax
from jax.experimental import pallas as pl
from jax.experimental.pallas import tpu as pltpu
from jax.experimental.pallas import tpu_sc as plsc

_VOCAB = 50257
_EMBED = 1024
_MAX_SEQ = 2048
_BATCH = 4
_NC = 2
_NS = 16
_NW = _NC * _NS
_SEQ_PER_W = _MAX_SEQ // _NW
_CHUNK = 16
_NQ = _SEQ_PER_W // _CHUNK
_NCHUNK = _BATCH * _NQ
_NBUF = 3
_LANES = 16


def _emb_body(ids_hbm, tok_hbm, pos_hbm, out_hbm,
              idx_v, tok0, tok1, tok2, posbuf, gsem, wsem, psem):
    wid = lax.axis_index("s") * _NC + lax.axis_index("c")
    seq0 = wid * _SEQ_PER_W

    pos_cp = pltpu.async_copy(pos_hbm.at[pl.ds(seq0, _SEQ_PER_W)], posbuf, psem)

    for b in range(_BATCH):
        pltpu.sync_copy(ids_hbm.at[b, wid], idx_v.at[b])

    tokbufs = (tok0, tok1, tok2)

    def chunk_bq(c):
        return divmod(c, _NQ)

    def fire_gather(c):
        b, q = chunk_bq(c)
        return pltpu.async_copy(
            tok_hbm.at[idx_v.at[b, q]], tokbufs[c % _NBUF], gsem)

    def add_rows(tokbuf, q):
        def row_body(r, _):
            for j in range(_EMBED // _LANES):
                s = pl.ds(j * _LANES, _LANES)
                tokbuf[r, s] = tokbuf[r, s] + posbuf[q * _CHUNK + r, s]
            return 0
        lax.fori_loop(0, _CHUNK, row_body, 0)

    gathers = [None] * _NBUF
    writes = [None] * _NBUF
    for k in range(_NBUF - 1):
        gathers[k] = fire_gather(k)

    pos_cp.wait()

    for c in range(_NCHUNK):
        b, q = chunk_bq(c)
        buf = c % _NBUF
        gathers[buf].wait()

        nxt = c + _NBUF - 1
        if nxt < _NCHUNK:
            nbuf = nxt % _NBUF
            if writes[nbuf] is not None:
                writes[nbuf].wait()
                writes[nbuf] = None
            gathers[nbuf] = fire_gather(nxt)

        add_rows(tokbufs[buf], q)

        row0 = b * _MAX_SEQ + seq0 + q * _CHUNK
        writes[buf] = pltpu.async_copy(
            tokbufs[buf], out_hbm.at[pl.ds(row0, _CHUNK)], wsem)

    for w in writes:
        if w is not None:
            w.wait()


@jax.jit
def _embed(ids, tok_table, pos_table):
    mesh = plsc.VectorSubcoreMesh(core_axis_name="c", subcore_axis_name="s")
    run = functools.partial(
        pl.kernel,
        out_type=jax.ShapeDtypeStruct((_BATCH * _MAX_SEQ, _EMBED), jnp.float32),
        mesh=mesh,
        scratch_types=[
            pltpu.VMEM((_BATCH, _NQ, _CHUNK), jnp.int32),
            pltpu.VMEM((_CHUNK, _EMBED), jnp.float32),
            pltpu.VMEM((_CHUNK, _EMBED), jnp.float32),
            pltpu.VMEM((_CHUNK, _EMBED), jnp.float32),
            pltpu.VMEM((_SEQ_PER_W, _EMBED), jnp.float32),
            pltpu.SemaphoreType.DMA,
            pltpu.SemaphoreType.DMA,
            pltpu.SemaphoreType.DMA,
        ],
    )(_emb_body)
    return run(ids, tok_table, pos_table)


def kernel(input_ids, token_table, pos_table):
    ids = input_ids.astype(jnp.int32).reshape(_BATCH, _NW, _NQ, _CHUNK)
    out = _embed(ids, token_table, pos_table)
    return out.reshape(_BATCH, _MAX_SEQ, _EMBED)


# speedup vs baseline: 1.2465x; 1.2465x over previous
import functools

import jax
import jax.numpy as jnp
from jax import lax
from jax.experimental import pallas as pl
from jax.experimental.pallas import tpu as pltpu
from jax.experimental.pallas import tpu_sc as plsc

_VOCAB = 50257
_EMBED = 1024
_MAX_SEQ = 2048
_BATCH = 4
_NC = 2
_NS = 16
_NW = _NC * _NS
_SEQ_PER_W = _MAX_SEQ // _NW
_CHUNK = 32
_NQ = _SEQ_PER_W // _CHUNK
_NCHUNK = _NQ * _BATCH
_LANES = 16


def _emb_body(ids_hbm, tok_hbm, pos_hbm, out_hbm,
              idx_v, tok0, tok1, posbuf, gsem, wsem):
    wid = lax.axis_index("s") * _NC + lax.axis_index("c")
    seq0 = wid * _SEQ_PER_W

    for b in range(_BATCH):
        pltpu.sync_copy(ids_hbm.at[b, wid], idx_v.at[b])

    tokbufs = (tok0, tok1)

    def fire_gather(c):
        q, b = divmod(c, _BATCH)
        return pltpu.async_copy(
            tok_hbm.at[idx_v.at[b, q]], tokbufs[c % 2], gsem)

    def add_rows(tokbuf):
        def row_body(r, _):
            for j in range(_EMBED // _LANES):
                s = pl.ds(j * _LANES, _LANES)
                tokbuf[r, s] = tokbuf[r, s] + posbuf[r, s]
            return 0
        lax.fori_loop(0, _CHUNK, row_body, 0)

    gathers = [fire_gather(0), None]
    writes = [None, None]

    for c in range(_NCHUNK):
        q, b = divmod(c, _BATCH)
        buf = c % 2

        if b == 0:
            pltpu.sync_copy(pos_hbm.at[pl.ds(seq0 + q * _CHUNK, _CHUNK)], posbuf)

        gathers[buf].wait()

        if c + 1 < _NCHUNK:
            nbuf = (c + 1) % 2
            if writes[nbuf] is not None:
                writes[nbuf].wait()
                writes[nbuf] = None
            gathers[nbuf] = fire_gather(c + 1)

        add_rows(tokbufs[buf])

        row0 = b * _MAX_SEQ + seq0 + q * _CHUNK
        writes[buf] = pltpu.async_copy(
            tokbufs[buf], out_hbm.at[pl.ds(row0, _CHUNK)], wsem)

    for w in writes:
        if w is not None:
            w.wait()


@jax.jit
def _embed(ids, tok_table, pos_table):
    mesh = plsc.VectorSubcoreMesh(core_axis_name="c", subcore_axis_name="s")
    run = functools.partial(
        pl.kernel,
        out_type=jax.ShapeDtypeStruct((_BATCH * _MAX_SEQ, _EMBED), jnp.float32),
        mesh=mesh,
        scratch_types=[
            pltpu.VMEM((_BATCH, _NQ, _CHUNK), jnp.int32),
            pltpu.VMEM((_CHUNK, _EMBED), jnp.float32),
            pltpu.VMEM((_CHUNK, _EMBED), jnp.float32),
            pltpu.VMEM((_CHUNK, _EMBED), jnp.float32),
            pltpu.SemaphoreType.DMA,
            pltpu.SemaphoreType.DMA,
        ],
    )(_emb_body)
    return run(ids, tok_table, pos_table)


def kernel(input_ids, token_table, pos_table):
    ids = input_ids.astype(jnp.int32).reshape(_BATCH, _NW, _NQ, _CHUNK)
    out = _embed(ids, token_table, pos_table)
    return out.reshape(_BATCH, _MAX_SEQ, _EMBED)
